# baseline (speedup 1.0000x reference)
import functools

import jax
import jax.numpy as jnp
from jax import lax
from jax.experimental import pallas as pl
from jax.experimental.pallas import tpu as pltpu
from jax.experimental.pallas import tpu_sc as plsc

_NC, _NS, _L = 2, 16, 16
_NW = _NC * _NS
_P = 128
_YB = 64


@functools.partial(jax.jit, static_argnums=(1, 2, 3, 4))
def _build_pair_table(x, B, C, H, W):
    M = B * H * W
    R = _YB * W

    def body(x_ref, out_ref):
        ident = (lax.broadcasted_iota(jnp.int32, (C, C), 0)
                 == lax.broadcasted_iota(jnp.int32, (C, C), 1)).astype(jnp.float32)
        a = x_ref[0].reshape(C, R)
        t = lax.dot_general(a, ident, (((0,), (0,)), ((), ())),
                            preferred_element_type=jnp.float32)
        shifted = jnp.concatenate([t[1:], t[:1]], axis=0)
        out_ref[...] = jnp.concatenate([t, shifted], axis=1)

    return pl.pallas_call(
        body,
        grid=(B, H // _YB),
        in_specs=[pl.BlockSpec((1, C, _YB, W), lambda b, h: (b, 0, h, 0))],
        out_specs=pl.BlockSpec((R, 2 * C), lambda b, h: (b * (H // _YB) + h, 0)),
        out_shape=jax.ShapeDtypeStruct((M, 2 * C), jnp.float32),
    )(x)


@functools.partial(jax.jit, static_argnums=(3, 4, 5, 6, 7))
def _sc_interp(pair_table, posx, posy, B, C, H, W, N):
    NPAD = -(-N // _P) * _P
    cpb = NPAD // _P
    wpb = _NW // B
    assert _NW % B == 0 and C % _L == 0
    per_w = -(-cpb // wpb)
    per_w += per_w % 2
    assert per_w * (wpb - 1) >= cpb - per_w
    HW = H * W
    fH, fW = float(H), float(W)
    sx = float(max(W - 1, 1))
    sy = float(max(H - 1, 1))
    mesh = plsc.VectorSubcoreMesh(core_axis_name="c", subcore_axis_name="s")

    @functools.partial(
        pl.kernel,
        out_type=jax.ShapeDtypeStruct((B, C, NPAD), jnp.float32),
        mesh=mesh,
        scratch_types=dict(
            px_v=pltpu.VMEM((4 * _P,), jnp.float32),
            py_v=pltpu.VMEM((4 * _P,), jnp.float32),
            idx_a=[pltpu.VMEM((_P,), jnp.int32) for _ in range(2)],
            idx_b=[pltpu.VMEM((_P,), jnp.int32) for _ in range(2)],
            w_a=[pltpu.VMEM((_P,), jnp.float32) for _ in range(4)],
            w_b=[pltpu.VMEM((_P,), jnp.float32) for _ in range(4)],
            g_a=[pltpu.VMEM((_P, 2 * C), jnp.float32) for _ in range(2)],
            g_b=[pltpu.VMEM((_P, 2 * C), jnp.float32) for _ in range(2)],
            o_a=pltpu.VMEM((_P, C), jnp.float32),
            o_b=pltpu.VMEM((_P, C), jnp.float32),
            psem=pltpu.SemaphoreType.DMA,
            gsem_a=pltpu.SemaphoreType.DMA,
            gsem_b=pltpu.SemaphoreType.DMA,
            osem_a=pltpu.SemaphoreType.DMA,
            osem_b=pltpu.SemaphoreType.DMA,
        ),
    )
    def kern(table_hbm, posx_hbm, posy_hbm, out_hbm,
             px_v, py_v, idx_a, idx_b, w_a, w_b, g_a, g_b, o_a, o_b,
             psem, gsem_a, gsem_b, osem_a, osem_b):
        wid = lax.axis_index("s") * _NC + lax.axis_index("c")
        b = lax.div(wid, wpb)
        start = jnp.minimum(lax.rem(wid, wpb) * per_w, cpb - per_w)
        rowbase0 = b * (H * W)
        pos_off = b * N
        lane = lax.iota(jnp.int32, 16)

        def fire_pos_pair(cpair, poff):
            off = pos_off + cpair * _P
            pltpu.async_copy(posx_hbm.at[pl.ds(off, 2 * _P)],
                             px_v.at[pl.ds(poff, 2 * _P)], psem)
            pltpu.async_copy(posy_hbm.at[pl.ds(off, 2 * _P)],
                             py_v.at[pl.ds(poff, 2 * _P)], psem)

        def wait_pos_pair(cpair, poff):
            off = pos_off + cpair * _P
            pltpu.make_async_copy(posx_hbm.at[pl.ds(off, 2 * _P)],
                                  px_v.at[pl.ds(poff, 2 * _P)], psem).wait()
            pltpu.make_async_copy(posy_hbm.at[pl.ds(off, 2 * _P)],
                                  py_v.at[pl.ds(poff, 2 * _P)], psem).wait()

        def index_phase(c, pbase, idx_v, w_v):
            for j in range(_P // _L):
                s = pl.ds(j * _L, _L)
                sp = pl.ds(pbase + j * _L, _L)
                px = px_v[sp]
                py = py_v[sp]
                gx = 2.0 * (px / sx) - 1.0
                gy = 2.0 * (py / sy) - 1.0
                ix = ((gx + 1.0) * fW - 1.0) / 2.0
                iy = ((gy + 1.0) * fH - 1.0) / 2.0
                tx = ix.astype(jnp.int32)
                tx = jnp.where(tx.astype(jnp.float32) > ix, tx - 1, tx)
                ty = iy.astype(jnp.int32)
                ty = jnp.where(ty.astype(jnp.float32) > iy, ty - 1, ty)
                wx1 = ix - tx.astype(jnp.float32)
                wx0 = 1.0 - wx1
                wy1 = iy - ty.astype(jnp.float32)
                wy0 = 1.0 - wy1
                x1 = tx + 1
                y1 = ty + 1
                ax = (jnp.where((tx >= 0) & (tx < W), wx0, 0.0)
                      + jnp.where(tx == -1, wx1, 0.0))
                bx = jnp.where((tx >= 0) & (x1 < W), wx1, 0.0)
                ay0 = jnp.where((ty >= 0) & (ty < H), wy0, 0.0)
                ay1 = jnp.where((y1 >= 0) & (y1 < H), wy1, 0.0)
                x0c = jnp.clip(tx, 0, W - 1)
                y0c = jnp.clip(ty, 0, H - 1)
                y1c = jnp.clip(y1, 0, H - 1)
                idx_v[0][s] = rowbase0 + y0c * W + x0c
                idx_v[1][s] = rowbase0 + y1c * W + x0c
                w_v[0][s] = ax * ay0
                w_v[1][s] = bx * ay0
                w_v[2][s] = ax * ay1
                w_v[3][s] = bx * ay1

        def fire_gathers(idx_v, g_v, sem):
            pltpu.async_copy(table_hbm.at[idx_v[0]], g_v[0], sem)
            pltpu.async_copy(table_hbm.at[idx_v[1]], g_v[1], sem)

        def wait_gathers(idx_v, g_v, sem):
            pltpu.make_async_copy(table_hbm.at[idx_v[0]], g_v[0], sem).wait()
            pltpu.make_async_copy(table_hbm.at[idx_v[1]], g_v[1], sem).wait()

        def combine(g_v, w_v, o_v):
            def grp_body(jv, _):
                gbase = jv * _L
                sg = pl.ds(gbase, _L)
                wv = [w_v[k][sg] for k in range(4)]
                for ii in range(_L):
                    p = gbase + ii
                    a00, a10, a01, a11 = wv[0][ii], wv[1][ii], wv[2][ii], wv[3][ii]
                    for q in range(C // _L):
                        s0 = pl.ds(q * _L, _L)
                        s1 = pl.ds(C + q * _L, _L)
                        o_v[p, s0] = (g_v[0][p, s0] * a00 + g_v[0][p, s1] * a10
                                      + g_v[1][p, s0] * a01 + g_v[1][p, s1] * a11)
                return 0

            lax.fori_loop(0, _P // _L, grp_body, 0)

        def fire_out(c, o_v, sem):
            n0 = c * _P
            for ch in range(C):
                pltpu.async_copy(o_v.at[:, ch], out_hbm.at[b, ch, pl.ds(n0, _P)],
                                 sem)

        def wait_out(c, o_v, sem):
            n0 = c * _P
            for ch in range(C):
                pltpu.make_async_copy(o_v.at[:, ch],
                                      out_hbm.at[b, ch, pl.ds(n0, _P)],
                                      sem).wait()

        fire_pos_pair(start, 0)
        wait_pos_pair(start, 0)
        index_phase(start, 0, idx_a, w_a)
        fire_gathers(idx_a, g_a, gsem_a)

        n_pairs = per_w // 2

        def pair_body(i, _):
            ca = start + 2 * i
            cb = ca + 1
            poff_cur = lax.rem(i, 2) * (2 * _P)
            poff_nxt = 2 * _P - poff_cur

            @pl.when(i < n_pairs - 1)
            def _():
                fire_pos_pair(ca + 2, poff_nxt)

            index_phase(cb, poff_cur + _P, idx_b, w_b)
            fire_gathers(idx_b, g_b, gsem_b)
            wait_gathers(idx_a, g_a, gsem_a)

            @pl.when(i > 0)
            def _():
                wait_out(ca, o_a, osem_a)

            combine(g_a, w_a, o_a)
            fire_out(ca, o_a, osem_a)

            @pl.when(i < n_pairs - 1)
            def _():
                wait_pos_pair(ca + 2, poff_nxt)
                index_phase(ca + 2, poff_nxt, idx_a, w_a)
                fire_gathers(idx_a, g_a, gsem_a)

            wait_gathers(idx_b, g_b, gsem_b)

            @pl.when(i > 0)
            def _():
                wait_out(cb, o_b, osem_b)

            combine(g_b, w_b, o_b)
            fire_out(cb, o_b, osem_b)
            return 0

        lax.fori_loop(0, n_pairs, pair_body, 0)
        last = start + per_w - 2
        wait_out(last, o_a, osem_a)
        wait_out(last + 1, o_b, osem_b)

    return kern(pair_table, posx, posy)


def kernel(x, pos, height, width):
    B, C, H, W = x.shape
    N = pos.shape[1]
    pair_table = _build_pair_table(x, B, C, H, W)
    posx = jnp.pad(pos[:, :, 0].reshape(-1), (0, 2 * _P))
    posy = jnp.pad(pos[:, :, 1].reshape(-1), (0, 2 * _P))
    out = _sc_interp(pair_table, posx, posy, B, C, H, W, N)
    return jnp.transpose(out, (0, 2, 1))[:, :N, :]


# speedup vs baseline: 134.2549x; 134.2549x over previous
import functools

import jax
import jax.numpy as jnp
from jax import lax
from jax.experimental import pallas as pl
from jax.experimental.pallas import tpu as pltpu
from jax.experimental.pallas import tpu_sc as plsc

_NC, _NS, _L = 2, 16, 16
_NW = _NC * _NS
_P = 128
_YB = 64


@functools.partial(jax.jit, static_argnums=(1, 2, 3, 4))
def _build_pair_table(x, B, C, H, W):
    M = B * H * W
    R = _YB * W

    def body(x_ref, out_ref):
        ident = (lax.broadcasted_iota(jnp.int32, (C, C), 0)
                 == lax.broadcasted_iota(jnp.int32, (C, C), 1)).astype(jnp.float32)
        a = x_ref[0].reshape(C, R)
        t = lax.dot_general(a, ident, (((0,), (0,)), ((), ())),
                            preferred_element_type=jnp.float32)
        shifted = jnp.concatenate([t[1:], t[:1]], axis=0)
        out_ref[...] = jnp.concatenate([t, shifted], axis=1)

    return pl.pallas_call(
        body,
        grid=(B, H // _YB),
        in_specs=[pl.BlockSpec((1, C, _YB, W), lambda b, h: (b, 0, h, 0))],
        out_specs=pl.BlockSpec((R, 2 * C), lambda b, h: (b * (H // _YB) + h, 0)),
        out_shape=jax.ShapeDtypeStruct((M, 2 * C), jnp.float32),
    )(x)


@functools.partial(jax.jit, static_argnums=(3, 4, 5, 6, 7))
def _sc_interp(pair_table, posx, posy, B, C, H, W, N):
    NP = B * N
    n_chunks = NP // _P
    assert NP % _P == 0 and C % _L == 0
    per_w = -(-n_chunks // _NW)
    per_w += per_w % 2
    assert per_w * (_NW - 1) >= n_chunks - per_w
    HW = H * W
    fH, fW = float(H), float(W)
    sx = float(max(W - 1, 1))
    sy = float(max(H - 1, 1))
    mesh = plsc.VectorSubcoreMesh(core_axis_name="c", subcore_axis_name="s")

    @functools.partial(
        pl.kernel,
        out_type=jax.ShapeDtypeStruct((NP, C), jnp.float32),
        mesh=mesh,
        scratch_types=dict(
            px_v=pltpu.VMEM((4 * _P,), jnp.float32),
            py_v=pltpu.VMEM((4 * _P,), jnp.float32),
            idx_a=[pltpu.VMEM((_P,), jnp.int32) for _ in range(2)],
            idx_b=[pltpu.VMEM((_P,), jnp.int32) for _ in range(2)],
            w_a=[pltpu.VMEM((_P,), jnp.float32) for _ in range(4)],
            w_b=[pltpu.VMEM((_P,), jnp.float32) for _ in range(4)],
            g_a=[pltpu.VMEM((_P, 2 * C), jnp.float32) for _ in range(2)],
            g_b=[pltpu.VMEM((_P, 2 * C), jnp.float32) for _ in range(2)],
            o_a=pltpu.VMEM((_P, C), jnp.float32),
            o_b=pltpu.VMEM((_P, C), jnp.float32),
            psem=pltpu.SemaphoreType.DMA,
            gsem_a=pltpu.SemaphoreType.DMA,
            gsem_b=pltpu.SemaphoreType.DMA,
            osem_a=pltpu.SemaphoreType.DMA,
            osem_b=pltpu.SemaphoreType.DMA,
        ),
    )
    def kern(table_hbm, posx_hbm, posy_hbm, out_hbm,
             px_v, py_v, idx_a, idx_b, w_a, w_b, g_a, g_b, o_a, o_b,
             psem, gsem_a, gsem_b, osem_a, osem_b):
        wid = lax.axis_index("s") * _NC + lax.axis_index("c")
        start = jnp.minimum(wid * per_w, n_chunks - per_w)
        lane = lax.iota(jnp.int32, 16)

        def fire_pos_pair(cpair, poff):
            off = cpair * _P
            pltpu.async_copy(posx_hbm.at[pl.ds(off, 2 * _P)],
                             px_v.at[pl.ds(poff, 2 * _P)], psem)
            pltpu.async_copy(posy_hbm.at[pl.ds(off, 2 * _P)],
                             py_v.at[pl.ds(poff, 2 * _P)], psem)

        def wait_pos_pair(cpair, poff):
            off = cpair * _P
            pltpu.make_async_copy(posx_hbm.at[pl.ds(off, 2 * _P)],
                                  px_v.at[pl.ds(poff, 2 * _P)], psem).wait()
            pltpu.make_async_copy(posy_hbm.at[pl.ds(off, 2 * _P)],
                                  py_v.at[pl.ds(poff, 2 * _P)], psem).wait()

        def index_phase(c, pbase, idx_v, w_v):
            for j in range(_P // _L):
                s = pl.ds(j * _L, _L)
                sp = pl.ds(pbase + j * _L, _L)
                px = px_v[sp]
                py = py_v[sp]
                gx = 2.0 * (px / sx) - 1.0
                gy = 2.0 * (py / sy) - 1.0
                ix = ((gx + 1.0) * fW - 1.0) / 2.0
                iy = ((gy + 1.0) * fH - 1.0) / 2.0
                tx = ix.astype(jnp.int32)
                tx = jnp.where(tx.astype(jnp.float32) > ix, tx - 1, tx)
                ty = iy.astype(jnp.int32)
                ty = jnp.where(ty.astype(jnp.float32) > iy, ty - 1, ty)
                wx1 = ix - tx.astype(jnp.float32)
                wx0 = 1.0 - wx1
                wy1 = iy - ty.astype(jnp.float32)
                wy0 = 1.0 - wy1
                x1 = tx + 1
                y1 = ty + 1
                ax = (jnp.where((tx >= 0) & (tx < W), wx0, 0.0)
                      + jnp.where(tx == -1, wx1, 0.0))
                bx = jnp.where((tx >= 0) & (x1 < W), wx1, 0.0)
                ay0 = jnp.where((ty >= 0) & (ty < H), wy0, 0.0)
                ay1 = jnp.where((y1 >= 0) & (y1 < H), wy1, 0.0)
                x0c = jnp.clip(tx, 0, W - 1)
                y0c = jnp.clip(ty, 0, H - 1)
                y1c = jnp.clip(y1, 0, H - 1)
                p_vec = c * _P + j * _L + lane
                rowbase = jnp.zeros((16,), jnp.int32)
                for k in range(1, B):
                    rowbase = rowbase + jnp.where(p_vec >= k * N, HW, 0)
                idx_v[0][s] = rowbase + y0c * W + x0c
                idx_v[1][s] = rowbase + y1c * W + x0c
                w_v[0][s] = ax * ay0
                w_v[1][s] = bx * ay0
                w_v[2][s] = ax * ay1
                w_v[3][s] = bx * ay1

        def fire_gathers(idx_v, g_v, sem):
            pltpu.async_copy(table_hbm.at[idx_v[0]], g_v[0], sem)
            pltpu.async_copy(table_hbm.at[idx_v[1]], g_v[1], sem)

        def wait_gathers(idx_v, g_v, sem):
            pltpu.make_async_copy(table_hbm.at[idx_v[0]], g_v[0], sem).wait()
            pltpu.make_async_copy(table_hbm.at[idx_v[1]], g_v[1], sem).wait()

        def combine(g_v, w_v, o_v):
            def grp_body(jv, _):
                gbase = jv * _L
                sg = pl.ds(gbase, _L)
                wv = [w_v[k][sg] for k in range(4)]
                for ii in range(_L):
                    p = gbase + ii
                    a00, a10, a01, a11 = wv[0][ii], wv[1][ii], wv[2][ii], wv[3][ii]
                    for q in range(C // _L):
                        s0 = pl.ds(q * _L, _L)
                        s1 = pl.ds(C + q * _L, _L)
                        o_v[p, s0] = (g_v[0][p, s0] * a00 + g_v[0][p, s1] * a10
                                      + g_v[1][p, s0] * a01 + g_v[1][p, s1] * a11)
                return 0

            lax.fori_loop(0, _P // _L, grp_body, 0)

        def fire_out(c, o_v, sem):
            pltpu.async_copy(o_v, out_hbm.at[pl.ds(c * _P, _P)], sem)

        def wait_out(c, o_v, sem):
            pltpu.make_async_copy(o_v, out_hbm.at[pl.ds(c * _P, _P)], sem).wait()

        fire_pos_pair(start, 0)
        wait_pos_pair(start, 0)
        index_phase(start, 0, idx_a, w_a)
        fire_gathers(idx_a, g_a, gsem_a)

        n_pairs = per_w // 2

        def pair_body(i, _):
            ca = start + 2 * i
            cb = ca + 1
            poff_cur = lax.rem(i, 2) * (2 * _P)
            poff_nxt = 2 * _P - poff_cur

            @pl.when(i < n_pairs - 1)
            def _():
                fire_pos_pair(ca + 2, poff_nxt)

            index_phase(cb, poff_cur + _P, idx_b, w_b)
            fire_gathers(idx_b, g_b, gsem_b)
            wait_gathers(idx_a, g_a, gsem_a)

            @pl.when(i > 0)
            def _():
                wait_out(ca, o_a, osem_a)

            combine(g_a, w_a, o_a)
            fire_out(ca, o_a, osem_a)

            @pl.when(i < n_pairs - 1)
            def _():
                wait_pos_pair(ca + 2, poff_nxt)
                index_phase(ca + 2, poff_nxt, idx_a, w_a)
                fire_gathers(idx_a, g_a, gsem_a)

            wait_gathers(idx_b, g_b, gsem_b)

            @pl.when(i > 0)
            def _():
                wait_out(cb, o_b, osem_b)

            combine(g_b, w_b, o_b)
            fire_out(cb, o_b, osem_b)
            return 0

        lax.fori_loop(0, n_pairs, pair_body, 0)
        last = start + per_w - 2
        wait_out(last, o_a, osem_a)
        wait_out(last + 1, o_b, osem_b)

    return kern(pair_table, posx, posy)


def kernel(x, pos, height, width):
    B, C, H, W = x.shape
    N = pos.shape[1]
    pair_table = _build_pair_table(x, B, C, H, W)
    posx = pos[:, :, 0].reshape(-1)
    posy = pos[:, :, 1].reshape(-1)
    out = _sc_interp(pair_table, posx, posy, B, C, H, W, N)
    return out.reshape(B, N, C)
